# Initial kernel scaffold; baseline (speedup 1.0000x reference)
import functools

import jax
import jax.numpy as jnp
from jax import lax
from jax.experimental import pallas as pl
from jax.experimental.pallas import tpu as pltpu
from jax.experimental.pallas import tpu_sc as plsc

E = 320000
D = 128
NC = 2
NS = 16
NW = NC * NS
EPW = E // NW
CH = 80
NCHUNK = EPW // CH
NG = CH // 16


def _sc_body(acc_hbm, txn_hbm, src_hbm, dst_hbm, out_hbm,
             src_v, dst_v, rows_a, rows_t, out_v, sem_a, sem_t):
    wid = lax.axis_index("s") * NC + lax.axis_index("c")
    base = wid * EPW

    def chunk_body(i, _):
        off = base + i * CH
        pltpu.sync_copy(src_hbm.at[pl.ds(off, CH)], src_v)
        pltpu.sync_copy(dst_hbm.at[pl.ds(off, CH)], dst_v)
        cp_a = pltpu.async_copy(acc_hbm.at[src_v], rows_a, sem_a)
        cp_t = pltpu.async_copy(txn_hbm.at[dst_v], rows_t, sem_t)
        cp_a.wait()
        cp_t.wait()

        def group_body(g, _):
            eidx = g * 16 + lax.iota(jnp.int32, 16)

            def d_body(j, acc):
                for u in range(8):
                    dd = j * 8 + u
                    dsplat = jnp.full((16,), dd, jnp.int32)
                    va = plsc.load_gather(rows_a, [eidx, dsplat])
                    vt = plsc.load_gather(rows_t, [eidx, dsplat])
                    acc = acc + va * vt
                return acc

            acc = lax.fori_loop(0, D // 8, d_body, jnp.zeros((16,), jnp.float32))
            sig = 1.0 / (1.0 + jnp.exp(-acc))
            out_v[pl.ds(g * 16, 16)] = sig
            return 0

        lax.fori_loop(0, NG, group_body, 0)
        pltpu.sync_copy(out_v, out_hbm.at[pl.ds(off, CH)])
        return 0

    lax.fori_loop(0, NCHUNK, chunk_body, 0)


@jax.jit
def _run(acc_emb, txn_emb, src, dst):
    mesh = plsc.VectorSubcoreMesh(core_axis_name="c", subcore_axis_name="s")
    k = functools.partial(
        pl.kernel,
        mesh=mesh,
        out_type=jax.ShapeDtypeStruct((E,), jnp.float32),
        scratch_types=[
            pltpu.VMEM((CH,), jnp.int32),
            pltpu.VMEM((CH,), jnp.int32),
            pltpu.VMEM((CH, D), jnp.float32),
            pltpu.VMEM((CH, D), jnp.float32),
            pltpu.VMEM((CH,), jnp.float32),
            pltpu.SemaphoreType.DMA,
            pltpu.SemaphoreType.DMA,
        ],
    )(_sc_body)
    return k(acc_emb, txn_emb, src, dst)


def kernel(account_embeddings, transaction_embeddings, edge_index):
    src = edge_index[0].astype(jnp.int32)
    dst = edge_index[1].astype(jnp.int32)
    return _run(account_embeddings, transaction_embeddings, src, dst)


# speedup vs baseline: 1.0983x; 1.0983x over previous
import functools

import jax
import jax.numpy as jnp
from jax import lax
from jax.experimental import pallas as pl
from jax.experimental.pallas import tpu as pltpu
from jax.experimental.pallas import tpu_sc as plsc

E = 320000
D = 128
NC = 2
NS = 16
NW = NC * NS
EPW = E // NW
CH = 80
NCHUNK = EPW // CH
NG = CH // 16


def _sc_body(acc_hbm, txn_hbm, src_hbm, dst_hbm, out_hbm,
             src_v, dst_v, rows_a, rows_t, out_v, sem_a, sem_t):
    wid = lax.axis_index("s") * NC + lax.axis_index("c")
    base = wid * EPW

    def chunk_body(i, _):
        off = base + i * CH
        pltpu.sync_copy(src_hbm.at[pl.ds(off, CH)], src_v)
        pltpu.sync_copy(dst_hbm.at[pl.ds(off, CH)], dst_v)
        cp_a = pltpu.async_copy(acc_hbm.at[src_v], rows_a, sem_a)
        cp_t = pltpu.async_copy(txn_hbm.at[dst_v], rows_t, sem_t)
        cp_a.wait()
        cp_t.wait()

        def group_body(g, _):
            eidx = g * 16 + lax.iota(jnp.int32, 16)

            def d_body(j, acc):
                for u in range(8):
                    dd = j * 8 + u
                    dsplat = jnp.full((16,), dd, jnp.int32)
                    va = plsc.load_gather(rows_a, [eidx, dsplat])
                    vt = plsc.load_gather(rows_t, [eidx, dsplat])
                    acc = acc + va * vt
                return acc

            acc = lax.fori_loop(0, D // 8, d_body, jnp.zeros((16,), jnp.float32))
            sig = 1.0 / (1.0 + jnp.exp(-acc))
            out_v[pl.ds(g * 16, 16)] = sig
            return 0

        lax.fori_loop(0, NG, group_body, 0)
        pltpu.sync_copy(out_v, out_hbm.at[pl.ds(off, CH)])
        return 0

    lax.fori_loop(0, NCHUNK, chunk_body, 0)


@jax.jit
def _run(acc_emb, txn_emb, src, dst):
    mesh = plsc.VectorSubcoreMesh(core_axis_name="c", subcore_axis_name="s")
    k = functools.partial(
        pl.kernel,
        mesh=mesh,
        compiler_params=pltpu.CompilerParams(needs_layout_passes=False),
        out_type=jax.ShapeDtypeStruct((E,), jnp.float32),
        scratch_types=[
            pltpu.VMEM((CH,), jnp.int32),
            pltpu.VMEM((CH,), jnp.int32),
            pltpu.VMEM((CH, D), jnp.float32),
            pltpu.VMEM((CH, D), jnp.float32),
            pltpu.VMEM((CH,), jnp.float32),
            pltpu.SemaphoreType.DMA,
            pltpu.SemaphoreType.DMA,
        ],
    )(_sc_body)
    return k(acc_emb, txn_emb, src, dst)


def kernel(account_embeddings, transaction_embeddings, edge_index):
    src = edge_index[0].astype(jnp.int32)
    dst = edge_index[1].astype(jnp.int32)
    return _run(account_embeddings, transaction_embeddings, src, dst)


# speedup vs baseline: 1.2368x; 1.1262x over previous
import functools

import jax
import jax.numpy as jnp
from jax import lax
from jax.experimental import pallas as pl
from jax.experimental.pallas import tpu as pltpu
from jax.experimental.pallas import tpu_sc as plsc

E = 320000
D = 128
NC = 2
NS = 16
NW = NC * NS
EPW = E // NW
CH = 80
NCHUNK = EPW // CH
NG = CH // 16
DU = 8


def _sc_body(acc_hbm, txn_hbm, src_hbm, dst_hbm, out_hbm,
             src0, dst0, src1, dst1, ra0, rt0, ra1, rt1, out_v,
             sa0, st0, sa1, st1):
    wid = lax.axis_index("s") * NC + lax.axis_index("c")
    base = wid * EPW
    bufs = ((src0, dst0, ra0, rt0, sa0, st0),
            (src1, dst1, ra1, rt1, sa1, st1))

    def fetch(c, b):
        srcv, dstv, ra, rt, sa, st = bufs[b]
        off = base + c * CH
        pltpu.sync_copy(src_hbm.at[pl.ds(off, CH)], srcv)
        pltpu.sync_copy(dst_hbm.at[pl.ds(off, CH)], dstv)
        pltpu.make_async_copy(acc_hbm.at[srcv], ra, sa).start()
        pltpu.make_async_copy(txn_hbm.at[dstv], rt, st).start()

    def consume(i, b):
        srcv, dstv, ra, rt, sa, st = bufs[b]
        pltpu.make_async_copy(acc_hbm.at[srcv], ra, sa).wait()
        pltpu.make_async_copy(txn_hbm.at[dstv], rt, st).wait()

        def group_body(g, _):
            eidx = g * 16 + lax.iota(jnp.int32, 16)

            def d_body(j, acc):
                for u in range(DU):
                    dd = j * DU + u
                    dsplat = jnp.full((16,), dd, jnp.int32)
                    va = plsc.load_gather(ra, [eidx, dsplat])
                    vt = plsc.load_gather(rt, [eidx, dsplat])
                    acc = acc + va * vt
                return acc

            acc = lax.fori_loop(0, D // DU, d_body,
                                jnp.zeros((16,), jnp.float32))
            sig = 1.0 / (1.0 + jnp.exp(-acc))
            out_v[pl.ds(i * CH + g * 16, 16)] = sig
            return 0

        lax.fori_loop(0, NG, group_body, 0)

    fetch(0, 0)
    fetch(1, 1)

    def pair_body(k, _):
        i0 = k * 2
        for b in range(2):
            i = i0 + b
            consume(i, b)

            @pl.when(i + 2 < NCHUNK)
            def _():
                fetch(i + 2, b)
        return 0

    lax.fori_loop(0, (NCHUNK - 1) // 2, pair_body, 0)
    consume(NCHUNK - 1, (NCHUNK - 1) % 2)

    pltpu.sync_copy(out_v, out_hbm.at[pl.ds(base, EPW)])


@jax.jit
def _run(acc_emb, txn_emb, src, dst):
    mesh = plsc.VectorSubcoreMesh(core_axis_name="c", subcore_axis_name="s")
    k = functools.partial(
        pl.kernel,
        mesh=mesh,
        compiler_params=pltpu.CompilerParams(needs_layout_passes=False),
        out_type=jax.ShapeDtypeStruct((E,), jnp.float32),
        scratch_types=[
            pltpu.VMEM((CH,), jnp.int32),
            pltpu.VMEM((CH,), jnp.int32),
            pltpu.VMEM((CH,), jnp.int32),
            pltpu.VMEM((CH,), jnp.int32),
            pltpu.VMEM((CH, D), jnp.float32),
            pltpu.VMEM((CH, D), jnp.float32),
            pltpu.VMEM((CH, D), jnp.float32),
            pltpu.VMEM((CH, D), jnp.float32),
            pltpu.VMEM((EPW,), jnp.float32),
            pltpu.SemaphoreType.DMA,
            pltpu.SemaphoreType.DMA,
            pltpu.SemaphoreType.DMA,
            pltpu.SemaphoreType.DMA,
        ],
    )(_sc_body)
    return k(acc_emb, txn_emb, src, dst)


def kernel(account_embeddings, transaction_embeddings, edge_index):
    src = edge_index[0].astype(jnp.int32)
    dst = edge_index[1].astype(jnp.int32)
    return _run(account_embeddings, transaction_embeddings, src, dst)


# speedup vs baseline: 8.3469x; 6.7486x over previous
import functools

import jax
import jax.numpy as jnp
from jax import lax
from jax.experimental import pallas as pl
from jax.experimental.pallas import tpu as pltpu
from jax.experimental.pallas import tpu_sc as plsc

E = 320000
D = 128
NC = 2
NS = 16
NW = NC * NS
EPW = E // NW
CH = 80
NCHUNK = EPW // CH
NG = CH // 16
DU = 8


def _sc_body(acc_hbm, txn_hbm, src_hbm, dst_hbm, out_hbm,
             src0, dst0, src1, dst1, ra0, rt0, ra1, rt1, out_v,
             sa0, st0, sa1, st1):
    wid = lax.axis_index("s") * NC + lax.axis_index("c")
    base = wid * EPW
    bufs = ((src0, dst0, ra0, rt0, sa0, st0),
            (src1, dst1, ra1, rt1, sa1, st1))

    def fetch(c, b):
        srcv, dstv, ra, rt, sa, st = bufs[b]
        off = base + c * CH
        pltpu.sync_copy(src_hbm.at[pl.ds(off, CH)], srcv)
        pltpu.sync_copy(dst_hbm.at[pl.ds(off, CH)], dstv)
        pltpu.make_async_copy(acc_hbm.at[srcv], ra, sa).start()
        pltpu.make_async_copy(txn_hbm.at[dstv], rt, st).start()

    def consume(i, b):
        srcv, dstv, ra, rt, sa, st = bufs[b]
        pltpu.make_async_copy(acc_hbm.at[srcv], ra, sa).wait()
        pltpu.make_async_copy(txn_hbm.at[dstv], rt, st).wait()

        def group_body(g, _):
            sig = ra[0, 0:16] + rt[0, 0:16]
            out_v[pl.ds(i * CH + g * 16, 16)] = sig
            return 0

        lax.fori_loop(0, NG, group_body, 0)

    fetch(0, 0)
    fetch(1, 1)

    def pair_body(k, _):
        i0 = k * 2
        for b in range(2):
            i = i0 + b
            consume(i, b)

            @pl.when(i + 2 < NCHUNK)
            def _():
                fetch(i + 2, b)
        return 0

    lax.fori_loop(0, (NCHUNK - 1) // 2, pair_body, 0)
    consume(NCHUNK - 1, (NCHUNK - 1) % 2)

    pltpu.sync_copy(out_v, out_hbm.at[pl.ds(base, EPW)])


@jax.jit
def _run(acc_emb, txn_emb, src, dst):
    mesh = plsc.VectorSubcoreMesh(core_axis_name="c", subcore_axis_name="s")
    k = functools.partial(
        pl.kernel,
        mesh=mesh,
        compiler_params=pltpu.CompilerParams(needs_layout_passes=False),
        out_type=jax.ShapeDtypeStruct((E,), jnp.float32),
        scratch_types=[
            pltpu.VMEM((CH,), jnp.int32),
            pltpu.VMEM((CH,), jnp.int32),
            pltpu.VMEM((CH,), jnp.int32),
            pltpu.VMEM((CH,), jnp.int32),
            pltpu.VMEM((CH, D), jnp.float32),
            pltpu.VMEM((CH, D), jnp.float32),
            pltpu.VMEM((CH, D), jnp.float32),
            pltpu.VMEM((CH, D), jnp.float32),
            pltpu.VMEM((EPW,), jnp.float32),
            pltpu.SemaphoreType.DMA,
            pltpu.SemaphoreType.DMA,
            pltpu.SemaphoreType.DMA,
            pltpu.SemaphoreType.DMA,
        ],
    )(_sc_body)
    return k(acc_emb, txn_emb, src, dst)


def kernel(account_embeddings, transaction_embeddings, edge_index):
    src = edge_index[0].astype(jnp.int32)
    dst = edge_index[1].astype(jnp.int32)
    return _run(account_embeddings, transaction_embeddings, src, dst)


# speedup vs baseline: 11.6808x; 1.3994x over previous
import functools

import jax
import jax.numpy as jnp
from jax import lax
from jax.experimental import pallas as pl
from jax.experimental.pallas import tpu as pltpu
from jax.experimental.pallas import tpu_sc as plsc

E = 320000
D = 128
NC = 2
NS = 16
NW = NC * NS
EPW = E // NW
CH = 80
NCHUNK = EPW // CH
NG = CH // 16
DU = 8
NBUF = 4


def _sc_body(acc_hbm, txn_hbm, src_hbm, dst_hbm, out_hbm,
             src_v, dst_v, out_v,
             ra0, rt0, ra1, rt1, ra2, rt2, ra3, rt3,
             sa0, st0, sa1, st1, sa2, st2, sa3, st3, sem_idx):
    wid = lax.axis_index("s") * NC + lax.axis_index("c")
    base = wid * EPW
    bufs = ((ra0, rt0, sa0, st0), (ra1, rt1, sa1, st1),
            (ra2, rt2, sa2, st2), (ra3, rt3, sa3, st3))

    cp_s = pltpu.make_async_copy(src_hbm.at[pl.ds(base, EPW)], src_v, sem_idx)
    cp_d = pltpu.make_async_copy(dst_hbm.at[pl.ds(base, EPW)], dst_v, sem_idx)
    cp_s.start()
    cp_d.start()
    cp_s.wait()
    cp_d.wait()

    def fetch(c, b):
        ra, rt, sa, st = bufs[b]
        pltpu.make_async_copy(
            acc_hbm.at[src_v.at[pl.ds(c * CH, CH)]], ra, sa).start()
        pltpu.make_async_copy(
            txn_hbm.at[dst_v.at[pl.ds(c * CH, CH)]], rt, st).start()

    def consume(i, b):
        ra, rt, sa, st = bufs[b]
        pltpu.make_async_copy(
            acc_hbm.at[src_v.at[pl.ds(i * CH, CH)]], ra, sa).wait()
        pltpu.make_async_copy(
            txn_hbm.at[dst_v.at[pl.ds(i * CH, CH)]], rt, st).wait()

        def group_body(g, _):
            eidx = g * 16 + lax.iota(jnp.int32, 16)

            def d_body(j, carry):
                acc, dvec = carry
                for _ in range(DU):
                    va = plsc.load_gather(ra, [eidx, dvec])
                    vt = plsc.load_gather(rt, [eidx, dvec])
                    acc = acc + va * vt
                    dvec = jnp.bitwise_and(dvec + 1, D - 1)
                return (acc, dvec)

            acc, _ = lax.fori_loop(
                0, D // DU, d_body,
                (jnp.zeros((16,), jnp.float32), lax.iota(jnp.int32, 16)))
            sig = 1.0 / (1.0 + jnp.exp(-acc))
            out_v[pl.ds(i * CH + g * 16, 16)] = sig
            return 0

        lax.fori_loop(0, NG, group_body, 0)

    for b in range(NBUF):
        fetch(b, b)

    def ring_body(k, _):
        i0 = k * NBUF
        for b in range(NBUF):
            i = i0 + b
            consume(i, b)

            @pl.when(i + NBUF < NCHUNK)
            def _():
                fetch(i + NBUF, b)
        return 0

    lax.fori_loop(0, (NCHUNK - 1) // NBUF, ring_body, 0)
    consume(NCHUNK - 1, (NCHUNK - 1) % NBUF)

    pltpu.sync_copy(out_v, out_hbm.at[pl.ds(base, EPW)])


@jax.jit
def _run(acc_emb, txn_emb, src, dst):
    mesh = plsc.VectorSubcoreMesh(core_axis_name="c", subcore_axis_name="s")
    k = functools.partial(
        pl.kernel,
        mesh=mesh,
        compiler_params=pltpu.CompilerParams(needs_layout_passes=False),
        out_type=jax.ShapeDtypeStruct((E,), jnp.float32),
        scratch_types=[
            pltpu.VMEM((EPW,), jnp.int32),
            pltpu.VMEM((EPW,), jnp.int32),
            pltpu.VMEM((EPW,), jnp.float32),
        ] + [pltpu.VMEM((CH, D), jnp.float32)] * (2 * NBUF)
          + [pltpu.SemaphoreType.DMA] * (2 * NBUF + 1),
    )(_sc_body)
    return k(acc_emb, txn_emb, src, dst)


def kernel(account_embeddings, transaction_embeddings, edge_index):
    src = edge_index[0].astype(jnp.int32)
    dst = edge_index[1].astype(jnp.int32)
    return _run(account_embeddings, transaction_embeddings, src, dst)
